# baseline (speedup 1.0000x reference)
import jax
import jax.numpy as jnp
from jax import lax
from jax.experimental import pallas as pl
from jax.experimental.pallas import tpu as pltpu
from jax.experimental.pallas import tpu_sc as plsc

VOCAB = 1000000
MAX_LEN = 200
D = 64
B = 1024
T = 200

NC = 2
NS = 16
NW = NC * NS
CH = 100
CPW = (B * T) // (NW * CH)
NBUF = 4
LANES = 16
VPR = D // LANES


def _sc_body(table, ids, pos, out, idx_v, pos_v, b0, b1, b2, b3,
             g0, g1, g2, g3, s0, s1, s2, s3):
    bufs = (b0, b1, b2, b3)
    gsem = (g0, g1, g2, g3)
    ssem = (s0, s1, s2, s3)
    wid = lax.axis_index("s") * NC + lax.axis_index("c")
    row0 = wid * CPW
    out0 = wid * CPW * CH

    pltpu.sync_copy(ids.at[pl.ds(row0, CPW)], idx_v)
    pltpu.sync_copy(pos, pos_v)

    def gather(s, b):
        pltpu.async_copy(table.at[idx_v.at[s]], bufs[b], gsem[b])

    def wait_gather(s, b):
        pltpu.make_async_copy(table.at[idx_v.at[s]], bufs[b], gsem[b]).wait()

    def store(s, b):
        pltpu.async_copy(bufs[b], out.at[pl.ds(out0 + s * CH, CH)], ssem[b])

    def wait_store(s, b):
        pltpu.make_async_copy(
            bufs[b], out.at[pl.ds(out0 + s * CH, CH)], ssem[b]).wait()

    for s in range(NBUF - 1):
        gather(s, s)

    def group(i, carry):
        g = i * NBUF
        for b in range(NBUF):
            s = g + b
            wait_gather(s, b)

            off = (b % 2) * CH

            def addpos(r, c, _b=b, _off=off):
                for v in range(VPR):
                    sl = pl.ds(v * LANES, LANES)
                    bufs[_b][r, sl] = bufs[_b][r, sl] + pos_v[_off + r, sl]
                return c
            lax.fori_loop(0, CH, addpos, 0, unroll=2)

            t = s + NBUF - 1
            tb = (b + NBUF - 1) % NBUF

            @pl.when(t < CPW)
            def _():
                @pl.when(t >= NBUF)
                def _():
                    wait_store(t - NBUF, tb)
                gather(t, tb)

            store(s, b)
        return carry

    lax.fori_loop(0, CPW // NBUF, group, 0)

    for s in range(CPW - NBUF, CPW):
        wait_store(s, s % NBUF)


def kernel(input_ids, token_emb, pos_emb):
    ids2d = input_ids.reshape(NW * CPW, CH).astype(jnp.int32)
    mesh = plsc.VectorSubcoreMesh(core_axis_name="c", subcore_axis_name="s")
    out = pl.kernel(
        _sc_body,
        out_type=jax.ShapeDtypeStruct((B * T, D), jnp.float32),
        mesh=mesh,
        compiler_params=pltpu.CompilerParams(use_tc_tiling_on_sc=False),
        scratch_types=[
            pltpu.VMEM((CPW, CH), jnp.int32),
            pltpu.VMEM((MAX_LEN, D), jnp.float32),
        ] + [pltpu.VMEM((CH, D), jnp.float32) for _ in range(NBUF)]
          + [pltpu.SemaphoreType.DMA for _ in range(2 * NBUF)],
    )(token_emb, ids2d, pos_emb)
    return out.reshape(B, T, D)
